# baseline (speedup 1.0000x reference)
import functools

import jax
import jax.numpy as jnp
from jax.experimental import pallas as pl
from jax.experimental.pallas import tpu as pltpu

_TOPK = 64


def _encoder_body(x_ref, w_ref, benc_ref, bdec_ref, hi_ref, lob_ref):
    w = w_ref[...]
    corr = jax.lax.dot_general(
        bdec_ref[...], w,
        dimension_numbers=(((1,), (1,)), ((), ())),
        preferred_element_type=jnp.float32,
    )
    acts = jax.lax.dot_general(
        x_ref[...], w,
        dimension_numbers=(((1,), (1,)), ((), ())),
        preferred_element_type=jnp.float32,
    )
    p = jnp.maximum(acts - corr + benc_ref[...], 0.0)
    bits = jax.lax.bitcast_convert_type(p, jnp.int32)
    hi_ref[...] = jax.lax.shift_right_arithmetic(
        bits, jnp.int32(16)).astype(jnp.int16)
    lob_ref[...] = ((bits & jnp.int32(0xFFFF))
                    - jnp.int32(32768)).astype(jnp.int16)


def _threshold_body(hi_ref, lob_ref, t_ref, *, k: int):
    rows = hi_ref.shape[0]
    ftot = hi_ref.shape[1]
    cw = 128
    nchunks = ftot // cw

    def hi_bit_step(i, ta):
        trial = ta | (jnp.int32(1) << (jnp.int32(14) - i))
        trial16 = trial.astype(jnp.int16)
        acc = jnp.zeros((rows, cw), jnp.int16)
        for j in range(nchunks):
            blk = hi_ref[:, j * cw:(j + 1) * cw]
            acc = acc + (blk >= trial16).astype(jnp.int16)
        cnt = jnp.sum(acc.astype(jnp.int32), axis=1, keepdims=True)
        return jnp.where(cnt >= k, trial, ta)

    ta = jax.lax.fori_loop(0, 15, hi_bit_step,
                           jnp.zeros((rows, 1), jnp.int32))
    ta16 = ta.astype(jnp.int16)

    acc = jnp.zeros((rows, cw), jnp.int16)
    for j in range(nchunks):
        blk = hi_ref[:, j * cw:(j + 1) * cw]
        acc = acc + (blk > ta16).astype(jnp.int16)
    c1 = jnp.sum(acc.astype(jnp.int32), axis=1, keepdims=True)

    def lo_bit_step(i, tb):
        utrial = tb | (jnp.int32(1) << (jnp.int32(15) - i))
        trial16 = (utrial - jnp.int32(32768)).astype(jnp.int16)
        acc = jnp.zeros((rows, cw), jnp.int16)
        for j in range(nchunks):
            hblk = hi_ref[:, j * cw:(j + 1) * cw]
            lblk = lob_ref[:, j * cw:(j + 1) * cw]
            acc = acc + ((hblk == ta16) &
                         (lblk >= trial16)).astype(jnp.int16)
        cnt = c1 + jnp.sum(acc.astype(jnp.int32), axis=1, keepdims=True)
        return jnp.where(cnt >= k, utrial, tb)

    tb = jax.lax.fori_loop(0, 16, lo_bit_step,
                           jnp.zeros((rows, 1), jnp.int32))

    t = jax.lax.shift_left(ta, jnp.int32(16)) | tb
    t_ref[...] = jnp.broadcast_to(t, t_ref.shape)


def _decode_body(hi_ref, lob_ref, wd_ref, t_ref, bdec_ref, out_ref):
    f = pl.program_id(0)
    t = t_ref[:, :1]
    hi = hi_ref[...].astype(jnp.int32)
    lo = lob_ref[...].astype(jnp.int32) + jnp.int32(32768)
    pbits = jax.lax.shift_left(hi, jnp.int32(16)) | lo
    e = jnp.where(pbits >= t,
                  jax.lax.bitcast_convert_type(pbits, jnp.float32), 0.0)
    contrib = jax.lax.dot_general(
        e, wd_ref[...],
        dimension_numbers=(((1,), (1,)), ((), ())),
        preferred_element_type=jnp.float32,
    )

    @pl.when(f == 0)
    def _init():
        out_ref[...] = bdec_ref[...] + contrib

    @pl.when(f > 0)
    def _acc():
        out_ref[...] += contrib


def kernel(x, W_enc, b_enc, W_dec, b_dec):
    B, D = x.shape
    F = W_enc.shape[0]
    benc2 = b_enc.reshape(1, F)
    bdec2 = b_dec.reshape(1, D)

    fb = min(1024, F)
    hi, lob = pl.pallas_call(
        _encoder_body,
        grid=(F // fb,),
        in_specs=[
            pl.BlockSpec((B, D), lambda f: (0, 0)),
            pl.BlockSpec((fb, D), lambda f: (f, 0)),
            pl.BlockSpec((1, fb), lambda f: (0, f)),
            pl.BlockSpec((1, D), lambda f: (0, 0)),
        ],
        out_specs=[
            pl.BlockSpec((B, fb), lambda f: (0, f)),
            pl.BlockSpec((B, fb), lambda f: (0, f)),
        ],
        out_shape=[
            jax.ShapeDtypeStruct((B, F), jnp.int16),
            jax.ShapeDtypeStruct((B, F), jnp.int16),
        ],
    )(x, W_enc, benc2, bdec2)

    rt = min(256, B)
    thresh = pl.pallas_call(
        functools.partial(_threshold_body, k=_TOPK),
        grid=(B // rt,),
        in_specs=[
            pl.BlockSpec((rt, F), lambda b: (b, 0)),
            pl.BlockSpec((rt, F), lambda b: (b, 0)),
        ],
        out_specs=pl.BlockSpec((rt, 128), lambda b: (b, 0)),
        out_shape=jax.ShapeDtypeStruct((B, 128), jnp.int32),
    )(hi, lob)

    fb2 = min(512, F)
    x_hat = pl.pallas_call(
        _decode_body,
        grid=(F // fb2,),
        in_specs=[
            pl.BlockSpec((B, fb2), lambda f: (0, f)),
            pl.BlockSpec((B, fb2), lambda f: (0, f)),
            pl.BlockSpec((D, fb2), lambda f: (0, f)),
            pl.BlockSpec((B, 128), lambda f: (0, 0)),
            pl.BlockSpec((1, D), lambda f: (0, 0)),
        ],
        out_specs=pl.BlockSpec((B, D), lambda f: (0, 0)),
        out_shape=jax.ShapeDtypeStruct((B, D), jnp.float32),
    )(hi, lob, W_dec, thresh, bdec2)
    return x_hat


# speedup vs baseline: 1.1368x; 1.1368x over previous
import functools

import jax
import jax.numpy as jnp
from jax.experimental import pallas as pl

_TOPK = 64


def _encoder_body(x_ref, w_ref, benc_ref, bdec_ref, out_ref):
    w = w_ref[...]
    corr = jax.lax.dot_general(
        bdec_ref[...], w,
        dimension_numbers=(((1,), (1,)), ((), ())),
        preferred_element_type=jnp.float32,
    )
    acts = jax.lax.dot_general(
        x_ref[...], w,
        dimension_numbers=(((1,), (1,)), ((), ())),
        preferred_element_type=jnp.float32,
    )
    out_ref[...] = jnp.maximum(acts - corr + benc_ref[...], 0.0)


def _threshold_body(p_ref, t_ref, *, k: int):
    rows = p_ref.shape[0]
    ftot = p_ref.shape[1]
    cw = 128
    nchunks = ftot // cw

    def bit_step(i, t):
        trial = t | (jnp.int32(1) << (jnp.int32(30) - i))
        acc = jnp.zeros((rows, cw), jnp.int32)
        for j in range(nchunks):
            blk = jax.lax.bitcast_convert_type(
                p_ref[:, j * cw:(j + 1) * cw], jnp.int32)
            acc = acc + (blk >= trial).astype(jnp.int32)
        cnt = jnp.sum(acc, axis=1, keepdims=True)
        return jnp.where(cnt >= k, trial, t)

    t = jax.lax.fori_loop(0, 31, bit_step, jnp.zeros((rows, 1), jnp.int32))
    t_ref[...] = jnp.broadcast_to(t, t_ref.shape)


def _decode_body(p_ref, wd_ref, t_ref, bdec_ref, out_ref):
    f = pl.program_id(0)
    t = t_ref[:, :1]
    pf = p_ref[...]
    pfbits = jax.lax.bitcast_convert_type(pf, jnp.int32)
    e = jnp.where(pfbits >= t, pf, 0.0)
    contrib = jax.lax.dot_general(
        e, wd_ref[...],
        dimension_numbers=(((1,), (1,)), ((), ())),
        preferred_element_type=jnp.float32,
    )

    @pl.when(f == 0)
    def _init():
        out_ref[...] = bdec_ref[...] + contrib

    @pl.when(f > 0)
    def _acc():
        out_ref[...] += contrib


def kernel(x, W_enc, b_enc, W_dec, b_dec):
    B, D = x.shape
    F = W_enc.shape[0]
    benc2 = b_enc.reshape(1, F)
    bdec2 = b_dec.reshape(1, D)

    fb = min(1024, F)
    preact = pl.pallas_call(
        _encoder_body,
        grid=(F // fb,),
        in_specs=[
            pl.BlockSpec((B, D), lambda f: (0, 0)),
            pl.BlockSpec((fb, D), lambda f: (f, 0)),
            pl.BlockSpec((1, fb), lambda f: (0, f)),
            pl.BlockSpec((1, D), lambda f: (0, 0)),
        ],
        out_specs=pl.BlockSpec((B, fb), lambda f: (0, f)),
        out_shape=jax.ShapeDtypeStruct((B, F), jnp.float32),
    )(x, W_enc, benc2, bdec2)

    rt = min(256, B)
    thresh = pl.pallas_call(
        functools.partial(_threshold_body, k=_TOPK),
        grid=(B // rt,),
        in_specs=[pl.BlockSpec((rt, F), lambda b: (b, 0))],
        out_specs=pl.BlockSpec((rt, 128), lambda b: (b, 0)),
        out_shape=jax.ShapeDtypeStruct((B, 128), jnp.int32),
    )(preact)

    fb2 = min(512, F)
    x_hat = pl.pallas_call(
        _decode_body,
        grid=(F // fb2,),
        in_specs=[
            pl.BlockSpec((B, fb2), lambda f: (0, f)),
            pl.BlockSpec((D, fb2), lambda f: (0, f)),
            pl.BlockSpec((B, 128), lambda f: (0, 0)),
            pl.BlockSpec((1, D), lambda f: (0, 0)),
        ],
        out_specs=pl.BlockSpec((B, D), lambda f: (0, 0)),
        out_shape=jax.ShapeDtypeStruct((B, D), jnp.float32),
    )(preact, W_dec, thresh, bdec2)
    return x_hat
